# baseline (speedup 1.0000x reference)
import functools

import jax
import jax.numpy as jnp
from jax import lax
from jax.experimental import pallas as pl
from jax.experimental.pallas import tpu as pltpu
from jax.experimental.pallas import tpu_sc as plsc

_NC = 2
_NS = 16


def _sc_month_gather(months_hbm, mt_hbm, me_hbm, midx_v, mrows_v, gsem):
    b = months_hbm.shape[0]
    wid = lax.axis_index("s") * _NC + lax.axis_index("c")

    @pl.when(wid < b)
    def _gather():
        pltpu.sync_copy(months_hbm.at[wid], midx_v)
        pltpu.async_copy(mt_hbm.at[midx_v], mrows_v, gsem).wait()
        pltpu.sync_copy(mrows_v, me_hbm.at[wid])


def _month_rows(months, month_table):
    b, t = months.shape
    n = month_table.shape[-1]
    runner = functools.partial(
        pl.kernel,
        out_type=jax.ShapeDtypeStruct((b, t, n), jnp.float32),
        mesh=plsc.VectorSubcoreMesh(
            core_axis_name="c", subcore_axis_name="s",
            num_cores=_NC, num_subcores=_NS),
        scratch_types=[
            pltpu.VMEM((t,), jnp.int32),
            pltpu.VMEM((t, n), jnp.float32),
            pltpu.SemaphoreType.DMA,
        ],
    )(_sc_month_gather)
    return runner(months, month_table)


def _tc_main_kernel(x_ref, ce_ref, pe_ref, o_ref):
    t = pe_ref.shape[0]
    n = ce_ref.shape[-1]
    x = x_ref[...]
    ce = ce_ref[...]
    pe = pe_ref[...]
    o_ref[..., 0:n] = x[..., 0:n] + ce[None, None, None, :, :]
    o_ref[..., n:2 * n] = x[..., n:2 * n] + pe[None, None, :, None, :]
    o_ref[..., 2 * n:] = x[..., 2 * n:]


def _tc_month_kernel(y_ref, me_ref, o_ref):
    o_ref[...] = y_ref[...] + me_ref[0, 0, 0, :][None, None, None, None, :]


def kernel(sensor_tokens, timestamps, channel_embed, pos_embed, month_table):
    b, h, w, t, b_s, d = sensor_tokens.shape
    n = d // 4
    hw = h * w
    br = 32
    x = sensor_tokens.reshape(b, hw, t, b_s, d)
    months = timestamps[:, :, 1].astype(jnp.int32)

    me = _month_rows(months, month_table)
    me4 = me.reshape(b, t, 1, n)

    y = pl.pallas_call(
        _tc_main_kernel,
        grid=(b, hw // br),
        in_specs=[
            pl.BlockSpec((1, br, t, b_s, d), lambda i, j: (i, j, 0, 0, 0)),
            pl.BlockSpec((b_s, n), lambda i, j: (0, 0)),
            pl.BlockSpec((t, n), lambda i, j: (0, 0)),
        ],
        out_specs=pl.BlockSpec((1, br, t, b_s, d), lambda i, j: (i, j, 0, 0, 0)),
        out_shape=jax.ShapeDtypeStruct(x.shape, x.dtype),
        compiler_params=pltpu.CompilerParams(
            dimension_semantics=("arbitrary", "arbitrary"),
        ),
    )(x, channel_embed, pos_embed[:t])

    out = pl.pallas_call(
        _tc_month_kernel,
        grid=(b, t),
        in_specs=[
            pl.BlockSpec((1, hw, 1, b_s, n), lambda i, j: (i, 0, j, 0, 2)),
            pl.BlockSpec((1, 1, 1, n), lambda i, j: (i, j, 0, 0)),
        ],
        out_specs=pl.BlockSpec((1, hw, 1, b_s, n), lambda i, j: (i, 0, j, 0, 2)),
        out_shape=jax.ShapeDtypeStruct(x.shape, x.dtype),
        input_output_aliases={0: 0},
        compiler_params=pltpu.CompilerParams(
            dimension_semantics=("arbitrary", "arbitrary"),
        ),
    )(y, me4)
    return out.reshape(b, h, w, t, b_s, d)


# speedup vs baseline: 1.2478x; 1.2478x over previous
import functools

import jax
import jax.numpy as jnp
from jax import lax
from jax.experimental import pallas as pl
from jax.experimental.pallas import tpu as pltpu
from jax.experimental.pallas import tpu_sc as plsc

_NC = 2
_NS = 16


def _sc_month_gather(months_hbm, mt_hbm, me_hbm, midx_v, mrows_v, gsem):
    b = months_hbm.shape[0]
    wid = lax.axis_index("s") * _NC + lax.axis_index("c")

    @pl.when(wid < b)
    def _gather():
        pltpu.sync_copy(months_hbm.at[wid], midx_v)
        pltpu.async_copy(mt_hbm.at[midx_v], mrows_v, gsem).wait()
        pltpu.sync_copy(mrows_v, me_hbm.at[wid])


def _month_rows(months, month_table):
    b, t = months.shape
    n = month_table.shape[-1]
    runner = functools.partial(
        pl.kernel,
        out_type=jax.ShapeDtypeStruct((b, t, n), jnp.float32),
        mesh=plsc.VectorSubcoreMesh(
            core_axis_name="c", subcore_axis_name="s",
            num_cores=_NC, num_subcores=_NS),
        scratch_types=[
            pltpu.VMEM((t,), jnp.int32),
            pltpu.VMEM((t, n), jnp.float32),
            pltpu.SemaphoreType.DMA,
        ],
    )(_sc_month_gather)
    return runner(months, month_table)


def _tc_add_kernel(x_ref, ce_ref, pe_ref, me_ref, o_ref):
    t = pe_ref.shape[0]
    n = ce_ref.shape[-1]
    x = x_ref[...]
    ce = ce_ref[...]
    pe = pe_ref[...]
    me = me_ref[0]
    o_ref[..., 0:n] = x[..., 0:n] + ce[None, None, None, :, :]
    o_ref[..., n:2 * n] = x[..., n:2 * n] + pe[None, None, :, None, :]
    o_ref[..., 2 * n:3 * n] = x[..., 2 * n:3 * n] + me[None, None, :, None, :]
    o_ref[..., 3 * n:] = x[..., 3 * n:]


def kernel(sensor_tokens, timestamps, channel_embed, pos_embed, month_table):
    b, h, w, t, b_s, d = sensor_tokens.shape
    n = d // 4
    hw = h * w
    br = 32
    x = sensor_tokens.reshape(b, hw, t, b_s, d)
    months = timestamps[:, :, 1].astype(jnp.int32)

    me = _month_rows(months, month_table)

    out = pl.pallas_call(
        _tc_add_kernel,
        grid=(b, hw // br),
        in_specs=[
            pl.BlockSpec((1, br, t, b_s, d), lambda i, j: (i, j, 0, 0, 0)),
            pl.BlockSpec((b_s, n), lambda i, j: (0, 0)),
            pl.BlockSpec((t, n), lambda i, j: (0, 0)),
            pl.BlockSpec((1, t, n), lambda i, j: (i, 0, 0)),
        ],
        out_specs=pl.BlockSpec((1, br, t, b_s, d), lambda i, j: (i, j, 0, 0, 0)),
        out_shape=jax.ShapeDtypeStruct(x.shape, x.dtype),
        compiler_params=pltpu.CompilerParams(
            dimension_semantics=("arbitrary", "arbitrary"),
        ),
    )(x, channel_embed, pos_embed[:t], me)
    return out.reshape(b, h, w, t, b_s, d)
